# baseline (speedup 1.0000x reference)
import functools

import jax
import jax.numpy as jnp
from jax import lax
from jax.experimental import pallas as pl
from jax.experimental.pallas import tpu as pltpu
from jax.experimental.pallas import tpu_sc as plsc

DIM_Z = 64
N_INT = 65
WPR = DIM_Z // 4
NC, NS = 2, 16
NW = NC * NS
LANES = 16
CHUNK = 128


def _make_sc_lookup(batch: int):
    bpw = batch // NW
    n_chunks = bpw // CHUNK
    mesh = plsc.VectorSubcoreMesh(core_axis_name="c", subcore_axis_name="s")

    @functools.partial(
        pl.kernel,
        mesh=mesh,
        out_type=jax.ShapeDtypeStruct((batch, WPR), jnp.int32),
        scratch_types=[
            pltpu.VMEM((bpw,), jnp.int32),
            pltpu.VMEM((N_INT,), jnp.int32),
            pltpu.VMEM((n_chunks, CHUNK), jnp.int32),
            pltpu.VMEM((bpw, WPR), jnp.int32),
            pltpu.SemaphoreType.DMA,
        ],
        compiler_params=pltpu.CompilerParams(
            needs_layout_passes=False, use_tc_tiling_on_sc=False),
    )
    def sc_lookup(labels_hbm, perm_hbm, masks_hbm, out_hbm,
                  labels_v, perm_v, idx_v, rows_v, sem):
        wid = lax.axis_index("s") * NC + lax.axis_index("c")
        base = wid * bpw
        pltpu.sync_copy(labels_hbm.at[pl.ds(base, bpw)], labels_v)
        pltpu.sync_copy(perm_hbm, perm_v)
        for i in range(bpw // LANES):
            lbl = labels_v[pl.ds(i * LANES, LANES)]
            iv = plsc.load_gather(perm_v, [lbl])
            idx_v[i // (CHUNK // LANES),
                  pl.ds((i % (CHUNK // LANES)) * LANES, LANES)] = iv
        copies = [
            pltpu.async_copy(masks_hbm.at[idx_v.at[j]],
                             rows_v.at[pl.ds(j * CHUNK, CHUNK)], sem)
            for j in range(n_chunks)
        ]
        for c in copies:
            c.wait()
        pltpu.sync_copy(rows_v, out_hbm.at[pl.ds(base, bpw)])

    return sc_lookup


def kernel(intervention_label, permutation, masks):
    batch = intervention_label.shape[0]
    masks_words = masks.reshape(N_INT, WPR, 4).view(jnp.int32)
    masks_words = masks_words.reshape(N_INT, WPR)
    words = _make_sc_lookup(batch)(intervention_label, permutation, masks_words)
    return words.view(jnp.bool_).reshape(batch, DIM_Z)


# speedup vs baseline: 1.1485x; 1.1485x over previous
import functools

import jax
import jax.numpy as jnp
from jax import lax
from jax.experimental import pallas as pl
from jax.experimental.pallas import tpu as pltpu
from jax.experimental.pallas import tpu_sc as plsc

DIM_Z = 64
N_INT = 65
WPR = DIM_Z // 4
NC, NS = 2, 16
NW = NC * NS
LANES = 16


def _make_sc_lookup(batch: int):
    bpw = batch // NW
    wpw = bpw * WPR
    mesh = plsc.VectorSubcoreMesh(core_axis_name="c", subcore_axis_name="s")

    @functools.partial(
        pl.kernel,
        mesh=mesh,
        out_type=jax.ShapeDtypeStruct((batch * WPR,), jnp.int32),
        scratch_types=[
            pltpu.VMEM((bpw,), jnp.int32),
            pltpu.VMEM((N_INT,), jnp.int32),
            pltpu.VMEM((N_INT * WPR,), jnp.int32),
            pltpu.VMEM((wpw,), jnp.int32),
            pltpu.SemaphoreType.DMA,
        ],
        compiler_params=pltpu.CompilerParams(
            needs_layout_passes=False, use_tc_tiling_on_sc=False),
    )
    def sc_lookup(labels_hbm, perm_hbm, masks_hbm, out_hbm,
                  labels_v, perm_v, table_v, rows_v, sem):
        wid = lax.axis_index("s") * NC + lax.axis_index("c")
        base = wid * bpw
        pltpu.sync_copy(labels_hbm.at[pl.ds(base, bpw)], labels_v)
        pltpu.sync_copy(perm_hbm, perm_v)
        pltpu.sync_copy(masks_hbm, table_v)
        lanes16 = lax.iota(jnp.int32, LANES) * WPR
        for i in range(bpw // LANES):
            lbl = labels_v[pl.ds(i * LANES, LANES)]
            idx = plsc.load_gather(perm_v, [lbl])
            idx16 = idx * WPR
            for w in range(WPR):
                vals = plsc.load_gather(table_v, [idx16 + w])
                plsc.store_scatter(
                    rows_v, [lanes16 + (i * LANES * WPR + w)], vals)
        pltpu.sync_copy(rows_v, out_hbm.at[pl.ds(wid * wpw, wpw)])

    return sc_lookup


def kernel(intervention_label, permutation, masks):
    batch = intervention_label.shape[0]
    masks_words = masks.reshape(N_INT * WPR, 4).view(jnp.int32).reshape(-1)
    words = _make_sc_lookup(batch)(intervention_label, permutation, masks_words)
    return words.view(jnp.bool_).reshape(batch, DIM_Z)


# speedup vs baseline: 1.3144x; 1.1444x over previous
import functools

import jax
import jax.numpy as jnp
from jax import lax
from jax.experimental import pallas as pl
from jax.experimental.pallas import tpu as pltpu
from jax.experimental.pallas import tpu_sc as plsc

DIM_Z = 64
N_INT = 65
WPR = DIM_Z // 4
NC, NS = 2, 16
NW = NC * NS
LANES = 16


def _make_sc_lookup(batch: int):
    bpw = batch // NW
    wpw = bpw * WPR
    mesh = plsc.VectorSubcoreMesh(core_axis_name="c", subcore_axis_name="s")

    @functools.partial(
        pl.kernel,
        mesh=mesh,
        out_type=jax.ShapeDtypeStruct((batch * WPR,), jnp.int32),
        scratch_types=[
            pltpu.VMEM((bpw,), jnp.int32),
            pltpu.VMEM((N_INT,), jnp.int32),
            pltpu.VMEM((N_INT * WPR,), jnp.int32),
            pltpu.VMEM((wpw,), jnp.int32),
            pltpu.SemaphoreType.DMA,
        ],
        compiler_params=pltpu.CompilerParams(
            needs_layout_passes=False, use_tc_tiling_on_sc=False),
    )
    def sc_lookup(labels_hbm, perm_hbm, masks_hbm, out_hbm,
                  labels_v, perm_v, table_v, rows_v, sem):
        wid = lax.axis_index("s") * NC + lax.axis_index("c")
        base = wid * bpw
        pltpu.sync_copy(labels_hbm.at[pl.ds(base, bpw)], labels_v)
        pltpu.sync_copy(perm_hbm, perm_v)
        pltpu.sync_copy(masks_hbm, table_v)
        lanes16 = lax.iota(jnp.int32, LANES) * WPR

        @plsc.parallel_loop(0, bpw // LANES, 1, unroll=4)
        def _groups(i):
            lbl = labels_v[pl.ds(i * LANES, LANES)]
            idx = plsc.load_gather(perm_v, [lbl])
            idx16 = idx * WPR
            pos = lanes16 + i * (LANES * WPR)
            for w in range(WPR):
                vals = plsc.load_gather(table_v, [idx16 + w])
                plsc.store_scatter(rows_v, [pos + w], vals)
        pltpu.sync_copy(rows_v, out_hbm.at[pl.ds(wid * wpw, wpw)])

    return sc_lookup


def kernel(intervention_label, permutation, masks):
    batch = intervention_label.shape[0]
    masks_words = masks.reshape(N_INT * WPR, 4).view(jnp.int32).reshape(-1)
    words = _make_sc_lookup(batch)(intervention_label, permutation, masks_words)
    return words.view(jnp.bool_).reshape(batch, DIM_Z)
